# Initial kernel scaffold; baseline (speedup 1.0000x reference)
import jax
import jax.numpy as jnp
from jax.experimental import pallas as pl


def kernel(feat_0, feat_1, feat_2, feat_3, feat_4, feat_5, feat_6, feat_7, feat_8, feat_9, feat_10, feat_11, feat_12, feat_13, feat_14, feat_15, feat_16, feat_17, feat_18, feat_19, feat_20, feat_21, feat_22, feat_23, feat_24, feat_25, E_0, E_1, E_2, E_3, E_4, E_5, E_6, E_7, E_8, E_9, E_10, E_11, E_12, E_13, E_14, E_15, E_16, E_17, E_18, E_19, E_20, E_21, E_22, E_23, E_24, E_25, W1, b1, W2, b2):
    raise NotImplementedError("write your pallas kernel here")


# speedup vs baseline: 2.6505x; 2.6505x over previous
import jax
import jax.numpy as jnp
from jax.experimental import pallas as pl

NUM_FIELDS = 26
BATCH = 16384
VOCAB = 100000
EMB = 16
H1 = 128
H2 = 64


def _mlp_body(x_ref, w1_ref, b1_ref, w2_ref, b2_ref, o_ref):
    h = jnp.dot(x_ref[...], w1_ref[...], preferred_element_type=jnp.float32)
    h = jnp.maximum(h + b1_ref[...], 0.0)
    o = jnp.dot(h, w2_ref[...], preferred_element_type=jnp.float32)
    o_ref[...] = jnp.maximum(o + b2_ref[...], 0.0)


def _mlp(x, w1, b1, w2, b2, bb=2048):
    d = NUM_FIELDS * EMB
    return pl.pallas_call(
        _mlp_body,
        grid=(BATCH // bb,),
        in_specs=[
            pl.BlockSpec((bb, d), lambda i: (i, 0)),
            pl.BlockSpec((d, H1), lambda i: (0, 0)),
            pl.BlockSpec((1, H1), lambda i: (0, 0)),
            pl.BlockSpec((H1, H2), lambda i: (0, 0)),
            pl.BlockSpec((1, H2), lambda i: (0, 0)),
        ],
        out_specs=pl.BlockSpec((bb, H2), lambda i: (i, 0)),
        out_shape=jax.ShapeDtypeStruct((BATCH, H2), jnp.float32),
    )(x, w1, b1, w2, b2)


def kernel(feat_0, feat_1, feat_2, feat_3, feat_4, feat_5, feat_6, feat_7,
           feat_8, feat_9, feat_10, feat_11, feat_12, feat_13, feat_14,
           feat_15, feat_16, feat_17, feat_18, feat_19, feat_20, feat_21,
           feat_22, feat_23, feat_24, feat_25,
           E_0, E_1, E_2, E_3, E_4, E_5, E_6, E_7, E_8, E_9, E_10, E_11,
           E_12, E_13, E_14, E_15, E_16, E_17, E_18, E_19, E_20, E_21,
           E_22, E_23, E_24, E_25,
           W1, b1, W2, b2):
    feats = [feat_0, feat_1, feat_2, feat_3, feat_4, feat_5, feat_6, feat_7,
             feat_8, feat_9, feat_10, feat_11, feat_12, feat_13, feat_14,
             feat_15, feat_16, feat_17, feat_18, feat_19, feat_20, feat_21,
             feat_22, feat_23, feat_24, feat_25]
    tables = [E_0, E_1, E_2, E_3, E_4, E_5, E_6, E_7, E_8, E_9, E_10, E_11,
              E_12, E_13, E_14, E_15, E_16, E_17, E_18, E_19, E_20, E_21,
              E_22, E_23, E_24, E_25]
    x = jnp.concatenate(
        [jnp.take(t, f, axis=0) for t, f in zip(tables, feats)], axis=-1)
    return _mlp(x, W1, b1.reshape(1, H1), W2, b2.reshape(1, H2))
